# baseline (speedup 1.0000x reference)
import functools

import jax
import jax.numpy as jnp
from jax import lax
from jax.experimental import pallas as pl
from jax.experimental.pallas import tpu as pltpu
from jax.experimental.pallas import tpu_sc as plsc

N_NODES = 10000
IN_DIM = 256
HID = 512
N_EDGES = 160000

NPAD = 10240
CW = 128
NCHUNK = HID // CW
NC, NS = 2, 16
NW = NC * NS
EP = 163840
EPT = EP // NW
EPS = EP // NS
EB = 128
NST = EPS // EB
NH = 5
NSH = NST // NH
MB = 256
GM = NPAD // MB
RPT = NPAD // NS


def _sc_mesh():
  return plsc.VectorSubcoreMesh(
      core_axis_name="c", subcore_axis_name="s", num_cores=NC, num_subcores=NS)


def _hist_body(dst_hbm, parts_hbm, dst_v, hist_v):
  c = lax.axis_index("c")
  s = lax.axis_index("s")
  wid = c * NS + s
  pltpu.sync_copy(dst_hbm.at[pl.ds(wid * EPT, EPT)], dst_v)

  def zb(i, _):
    hist_v[pl.ds(i * 16, 16)] = jnp.zeros((16,), jnp.float32)
    return 0
  lax.fori_loop(0, NPAD // 16, zb, 0)

  ones = jnp.ones((16,), jnp.float32)

  def hb(i, _):
    idx = dst_v[pl.ds(i * 16, 16)]
    plsc.addupdate_scatter(hist_v, [idx], ones)
    return 0
  lax.fori_loop(0, EPT // 16, hb, 0)
  pltpu.sync_copy(hist_v, parts_hbm.at[wid])


def _make_hist():
  return functools.partial(
      pl.kernel,
      out_type=jax.ShapeDtypeStruct((NW, NPAD), jnp.float32),
      mesh=_sc_mesh(),
      compiler_params=pltpu.CompilerParams(needs_layout_passes=False),
      scratch_types=[
          pltpu.VMEM((EPT,), jnp.int32),
          pltpu.VMEM((NPAD,), jnp.float32),
      ],
  )(_hist_body)


def _agg_body(xwp_hbm, src3_hbm, dst3_hbm, out_hbm,
              sidx, didx, buf0, buf1, acc, sem0, sem1):
  c = lax.axis_index("c")
  s = lax.axis_index("s")

  bufs = (buf0, buf1)
  sems = (sem0, sem1)
  zr = RPT // EB

  for ci in range(NCHUNK // NC):
    chunk = c * (NCHUNK // NC) + ci

    def zb(i, _):
      r = i // (CW // 16)
      k = i % (CW // 16)
      buf0[r, pl.ds(k * 16, 16)] = jnp.zeros((16,), jnp.float32)
      return 0
    lax.fori_loop(0, EB * (CW // 16), zb, 0)
    for z in range(zr):
      pltpu.sync_copy(buf0, acc.at[pl.ds((s * zr + z) * EB, EB)])
    plsc.subcore_barrier()

    for h in range(NH):
      pltpu.sync_copy(src3_hbm.at[s].at[pl.ds(h * NSH, NSH)], sidx)
      pltpu.sync_copy(dst3_hbm.at[s].at[pl.ds(h * NSH, NSH)], didx)

      for b in range(2):
        pltpu.async_copy(xwp_hbm.at[chunk].at[sidx.at[b]], bufs[b], sems[b])

      def step(t, _):
        i0 = t * 2
        for b in range(2):
          i = i0 + b
          pltpu.make_async_copy(
              xwp_hbm.at[chunk].at[sidx.at[i]], bufs[b], sems[b]).wait()
          pltpu.sync_copy(bufs[b], acc.at[didx.at[i]], add=True)

          @pl.when(i0 < NSH - 2)
          def _():
            pltpu.async_copy(xwp_hbm.at[chunk].at[sidx.at[i + 2]], bufs[b], sems[b])
        return 0
      lax.fori_loop(0, NSH // 2, step, 0)
    plsc.subcore_barrier()
    pltpu.sync_copy(acc.at[pl.ds(s * RPT, RPT)],
                    out_hbm.at[chunk].at[pl.ds(s * RPT, RPT)])


def _make_agg():
  return functools.partial(
      pl.kernel,
      out_type=jax.ShapeDtypeStruct((NCHUNK, NPAD, CW), jnp.float32),
      mesh=_sc_mesh(),
      compiler_params=pltpu.CompilerParams(needs_layout_passes=False),
      scratch_types=[
          pltpu.VMEM((NSH, EB), jnp.int32),
          pltpu.VMEM((NSH, EB), jnp.int32),
          pltpu.VMEM((EB, CW), jnp.float32),
          pltpu.VMEM((EB, CW), jnp.float32),
          pltpu.VMEM_SHARED((NPAD, CW), jnp.float32),
          pltpu.SemaphoreType.DMA,
          pltpu.SemaphoreType.DMA,
      ],
  )(_agg_body)


def _dis_of(parts):
  return lax.rsqrt(jnp.sum(parts, axis=1, keepdims=True) + 1.0)


def _mm1_body(x_ref, w_ref, parts_ref, out_ref):
  dis = _dis_of(parts_ref[...])
  xb = x_ref[...]
  for j in range(NCHUNK):
    wj = w_ref[:, j * CW:(j + 1) * CW]
    out_ref[j] = jnp.dot(xb, wj, preferred_element_type=jnp.float32) * dis


def _mm1(xpad, W1, parts):
  return pl.pallas_call(
      _mm1_body,
      grid=(GM,),
      in_specs=[
          pl.BlockSpec((MB, IN_DIM), lambda i: (i, 0)),
          pl.BlockSpec((IN_DIM, HID), lambda i: (0, 0)),
          pl.BlockSpec((MB, NW), lambda i: (i, 0)),
      ],
      out_specs=pl.BlockSpec((NCHUNK, MB, CW), lambda i: (0, i, 0)),
      out_shape=jax.ShapeDtypeStruct((NCHUNK, NPAD, CW), jnp.float32),
  )(xpad, W1, parts)


def _mm2_body(s1_ref, xw1_ref, parts_ref, b1_ref, w2_ref, out_ref):
  dis = _dis_of(parts_ref[...])
  hs = []
  for k in range(NCHUNK):
    hk = jnp.maximum(dis * (s1_ref[k] + xw1_ref[k]) + b1_ref[k], 0.0)
    hs.append(hk)
  for j in range(NCHUNK):
    acc = jnp.zeros((MB, CW), jnp.float32)
    for k in range(NCHUNK):
      acc = acc + jnp.dot(hs[k], w2_ref[k * CW:(k + 1) * CW, j * CW:(j + 1) * CW],
                          preferred_element_type=jnp.float32)
    out_ref[j] = acc * dis


def _mm2(s1, xw1p, parts, b1r, W2):
  return pl.pallas_call(
      _mm2_body,
      grid=(GM,),
      in_specs=[
          pl.BlockSpec((NCHUNK, MB, CW), lambda i: (0, i, 0)),
          pl.BlockSpec((NCHUNK, MB, CW), lambda i: (0, i, 0)),
          pl.BlockSpec((MB, NW), lambda i: (i, 0)),
          pl.BlockSpec((NCHUNK, CW), lambda i: (0, 0)),
          pl.BlockSpec((HID, HID), lambda i: (0, 0)),
      ],
      out_specs=pl.BlockSpec((NCHUNK, MB, CW), lambda i: (0, i, 0)),
      out_shape=jax.ShapeDtypeStruct((NCHUNK, NPAD, CW), jnp.float32),
  )(s1, xw1p, parts, b1r, W2)


def _fin_body(s2_ref, xw2_ref, parts_ref, b2_ref, out_ref):
  dis = _dis_of(parts_ref[...])
  for k in range(NCHUNK):
    out_ref[:, k * CW:(k + 1) * CW] = dis * (s2_ref[k] + xw2_ref[k]) + b2_ref[k]


def _fin(s2, xw2p, parts, b2r):
  return pl.pallas_call(
      _fin_body,
      grid=(GM,),
      in_specs=[
          pl.BlockSpec((NCHUNK, MB, CW), lambda i: (0, i, 0)),
          pl.BlockSpec((NCHUNK, MB, CW), lambda i: (0, i, 0)),
          pl.BlockSpec((MB, NW), lambda i: (i, 0)),
          pl.BlockSpec((NCHUNK, CW), lambda i: (0, 0)),
      ],
      out_specs=pl.BlockSpec((MB, HID), lambda i: (i, 0)),
      out_shape=jax.ShapeDtypeStruct((NPAD, HID), jnp.float32),
  )(s2, xw2p, parts, b2r)


def kernel(x, edge_index, W1, b1, W2, b2):
  src = edge_index[0].astype(jnp.int32)
  dst = edge_index[1].astype(jnp.int32)
  epad = EP - N_EDGES
  srcp = jnp.concatenate([src, jnp.zeros((epad,), jnp.int32)])
  dstp = jnp.concatenate([dst, jnp.full((epad,), NPAD - 1, jnp.int32)])
  order = jnp.argsort(dstp)
  srcp = srcp[order]
  dstp = dstp[order]
  src3 = srcp.reshape(NS, NST, EB)
  dst3 = dstp.reshape(NS, NST, EB)
  xpad = jnp.concatenate([x, jnp.zeros((NPAD - N_NODES, IN_DIM), x.dtype)])
  b1r = b1.reshape(NCHUNK, CW)
  b2r = b2.reshape(NCHUNK, CW)

  hist = _make_hist()
  agg = _make_agg()

  parts = jnp.transpose(hist(dstp))
  xw1p = _mm1(xpad, W1, parts)
  s1 = agg(xw1p, src3, dst3)
  xw2p = _mm2(s1, xw1p, parts, b1r, W2)
  s2 = agg(xw2p, src3, dst3)
  out = _fin(s2, xw2p, parts, b2r)
  return out[:N_NODES]


# speedup vs baseline: 1.2083x; 1.2083x over previous
import functools

import jax
import jax.numpy as jnp
from jax import lax
from jax.experimental import pallas as pl
from jax.experimental.pallas import tpu as pltpu
from jax.experimental.pallas import tpu_sc as plsc

N_NODES = 10000
IN_DIM = 256
HID = 512
N_EDGES = 160000

NPAD = 10240
CW = 128
NCHUNK = HID // CW
NC, NS = 2, 16
NW = NC * NS
EP = 163840
EPT = EP // NW
EPS = EP // NS
EB = 128
NST = EPS // EB
NH = 5
NSH = NST // NH
MB = 256
GM = NPAD // MB
RPT = NPAD // NS


def _sc_mesh():
  return plsc.VectorSubcoreMesh(
      core_axis_name="c", subcore_axis_name="s", num_cores=NC, num_subcores=NS)


def _hist_body(dst_hbm, parts_hbm, dst_v, hist_v):
  c = lax.axis_index("c")
  s = lax.axis_index("s")
  wid = c * NS + s
  pltpu.sync_copy(dst_hbm.at[pl.ds(wid * EPT, EPT)], dst_v)

  def zb(i, _):
    hist_v[pl.ds(i * 16, 16)] = jnp.zeros((16,), jnp.float32)
    return 0
  lax.fori_loop(0, NPAD // 16, zb, 0)

  ones = jnp.ones((16,), jnp.float32)

  def hb(i, _):
    idx = dst_v[pl.ds(i * 16, 16)]
    plsc.addupdate_scatter(hist_v, [idx], ones)
    return 0
  lax.fori_loop(0, EPT // 16, hb, 0)
  pltpu.sync_copy(hist_v, parts_hbm.at[wid])


def _make_hist():
  return functools.partial(
      pl.kernel,
      out_type=jax.ShapeDtypeStruct((NW, NPAD), jnp.float32),
      mesh=_sc_mesh(),
      compiler_params=pltpu.CompilerParams(needs_layout_passes=False),
      scratch_types=[
          pltpu.VMEM((EPT,), jnp.int32),
          pltpu.VMEM((NPAD,), jnp.float32),
      ],
  )(_hist_body)


def _agg_body(xwp_hbm, src3_hbm, dst3_hbm, out_hbm,
              sidx, didx, buf0, buf1, acc, sem0, sem1):
  c = lax.axis_index("c")
  s = lax.axis_index("s")

  bufs = (buf0, buf1)
  sems = (sem0, sem1)
  zr = RPT // EB

  for ci in range(NCHUNK // NC):
    chunk = c * (NCHUNK // NC) + ci

    def zb(i, _):
      r = i // (CW // 16)
      k = i % (CW // 16)
      buf0[r, pl.ds(k * 16, 16)] = jnp.zeros((16,), jnp.float32)
      return 0
    lax.fori_loop(0, EB * (CW // 16), zb, 0)
    for z in range(zr):
      pltpu.sync_copy(buf0, acc.at[pl.ds((s * zr + z) * EB, EB)])
    plsc.subcore_barrier()

    for h in range(NH):
      pltpu.sync_copy(src3_hbm.at[s].at[pl.ds(h * NSH, NSH)], sidx)
      pltpu.sync_copy(dst3_hbm.at[s].at[pl.ds(h * NSH, NSH)], didx)

      for b in range(2):
        pltpu.async_copy(xwp_hbm.at[chunk].at[sidx.at[b]], bufs[b], sems[b])

      def step(t, _):
        i0 = t * 2
        for b in range(2):
          i = i0 + b
          pltpu.make_async_copy(
              xwp_hbm.at[chunk].at[sidx.at[i]], bufs[b], sems[b]).wait()
          pltpu.sync_copy(bufs[b], acc.at[didx.at[i]], add=True)

          @pl.when(i0 < NSH - 2)
          def _():
            pltpu.async_copy(xwp_hbm.at[chunk].at[sidx.at[i + 2]], bufs[b], sems[b])
        return 0
      lax.fori_loop(0, NSH // 2, step, 0)
    plsc.subcore_barrier()
    pltpu.sync_copy(acc.at[pl.ds(s * RPT, RPT)],
                    out_hbm.at[chunk].at[pl.ds(s * RPT, RPT)])


def _make_agg():
  return functools.partial(
      pl.kernel,
      out_type=jax.ShapeDtypeStruct((NCHUNK, NPAD, CW), jnp.float32),
      mesh=_sc_mesh(),
      compiler_params=pltpu.CompilerParams(needs_layout_passes=False),
      scratch_types=[
          pltpu.VMEM((NSH, EB), jnp.int32),
          pltpu.VMEM((NSH, EB), jnp.int32),
          pltpu.VMEM((EB, CW), jnp.float32),
          pltpu.VMEM((EB, CW), jnp.float32),
          pltpu.VMEM_SHARED((NPAD, CW), jnp.float32),
          pltpu.SemaphoreType.DMA,
          pltpu.SemaphoreType.DMA,
      ],
  )(_agg_body)


def _dis_of(parts):
  return lax.rsqrt(jnp.sum(parts, axis=1, keepdims=True) + 1.0)


def _mm1_body(x_ref, w_ref, parts_ref, out_ref):
  dis = _dis_of(parts_ref[...])
  xb = x_ref[...]
  for j in range(NCHUNK):
    wj = w_ref[:, j * CW:(j + 1) * CW]
    out_ref[j] = jnp.dot(xb, wj, preferred_element_type=jnp.float32) * dis


def _mm1(xpad, W1, parts):
  return pl.pallas_call(
      _mm1_body,
      grid=(GM,),
      in_specs=[
          pl.BlockSpec((MB, IN_DIM), lambda i: (i, 0)),
          pl.BlockSpec((IN_DIM, HID), lambda i: (0, 0)),
          pl.BlockSpec((MB, NW), lambda i: (i, 0)),
      ],
      out_specs=pl.BlockSpec((NCHUNK, MB, CW), lambda i: (0, i, 0)),
      out_shape=jax.ShapeDtypeStruct((NCHUNK, NPAD, CW), jnp.float32),
  )(xpad, W1, parts)


def _mm2_body(s1_ref, xw1_ref, parts_ref, b1_ref, w2_ref, out_ref):
  dis = _dis_of(parts_ref[...])
  hs = []
  for k in range(NCHUNK):
    hk = jnp.maximum(dis * (s1_ref[k] + xw1_ref[k]) + b1_ref[k], 0.0)
    hs.append(hk)
  for j in range(NCHUNK):
    acc = jnp.zeros((MB, CW), jnp.float32)
    for k in range(NCHUNK):
      acc = acc + jnp.dot(hs[k], w2_ref[k * CW:(k + 1) * CW, j * CW:(j + 1) * CW],
                          preferred_element_type=jnp.float32)
    out_ref[j] = acc * dis


def _mm2(s1, xw1p, parts, b1r, W2):
  return pl.pallas_call(
      _mm2_body,
      grid=(GM,),
      in_specs=[
          pl.BlockSpec((NCHUNK, MB, CW), lambda i: (0, i, 0)),
          pl.BlockSpec((NCHUNK, MB, CW), lambda i: (0, i, 0)),
          pl.BlockSpec((MB, NW), lambda i: (i, 0)),
          pl.BlockSpec((NCHUNK, CW), lambda i: (0, 0)),
          pl.BlockSpec((HID, HID), lambda i: (0, 0)),
      ],
      out_specs=pl.BlockSpec((NCHUNK, MB, CW), lambda i: (0, i, 0)),
      out_shape=jax.ShapeDtypeStruct((NCHUNK, NPAD, CW), jnp.float32),
  )(s1, xw1p, parts, b1r, W2)


def _fin_body(s2_ref, xw2_ref, parts_ref, b2_ref, out_ref):
  dis = _dis_of(parts_ref[...])
  for k in range(NCHUNK):
    out_ref[:, k * CW:(k + 1) * CW] = dis * (s2_ref[k] + xw2_ref[k]) + b2_ref[k]


def _fin(s2, xw2p, parts, b2r):
  return pl.pallas_call(
      _fin_body,
      grid=(GM,),
      in_specs=[
          pl.BlockSpec((NCHUNK, MB, CW), lambda i: (0, i, 0)),
          pl.BlockSpec((NCHUNK, MB, CW), lambda i: (0, i, 0)),
          pl.BlockSpec((MB, NW), lambda i: (i, 0)),
          pl.BlockSpec((NCHUNK, CW), lambda i: (0, 0)),
      ],
      out_specs=pl.BlockSpec((MB, HID), lambda i: (i, 0)),
      out_shape=jax.ShapeDtypeStruct((NPAD, HID), jnp.float32),
  )(s2, xw2p, parts, b2r)


def kernel(x, edge_index, W1, b1, W2, b2):
  src = edge_index[0].astype(jnp.int32)
  dst = edge_index[1].astype(jnp.int32)
  epad = EP - N_EDGES
  srcp = jnp.concatenate([src, jnp.zeros((epad,), jnp.int32)])
  dstp = jnp.concatenate([dst, jnp.full((epad,), NPAD - 1, jnp.int32)])
  src3 = srcp.reshape(NS, NST, EB)
  dst3 = dstp.reshape(NS, NST, EB)
  xpad = jnp.concatenate([x, jnp.zeros((NPAD - N_NODES, IN_DIM), x.dtype)])
  b1r = b1.reshape(NCHUNK, CW)
  b2r = b2.reshape(NCHUNK, CW)

  hist = _make_hist()
  agg = _make_agg()

  parts = jnp.transpose(hist(dstp))
  xw1p = _mm1(xpad, W1, parts)
  s1 = agg(xw1p, src3, dst3)
  xw2p = _mm2(s1, xw1p, parts, b1r, W2)
  s2 = agg(xw2p, src3, dst3)
  out = _fin(s2, xw2p, parts, b2r)
  return out[:N_NODES]
